# baseline (speedup 1.0000x reference)
import functools

import jax
import jax.numpy as jnp
from jax import lax
from jax.experimental import pallas as pl
from jax.experimental.pallas import tpu as pltpu
from jax.experimental.pallas import tpu_sc as plsc

BT = 256
_L = 16


def _block1_body(x_ref, w_ref, b_ref, g_ref, h_ref, gate_ref):
    h = jnp.dot(x_ref[...], w_ref[...], preferred_element_type=jnp.float32)
    h = jnp.maximum(h + b_ref[...][None, :], 0.0)
    h_ref[...] = h
    gate_ref[...] = jnp.dot(h, g_ref[...], preferred_element_type=jnp.float32)


def _block1(x, w, b, g, off, nrows):
    B, D = x.shape
    N = w.shape[1]
    off_t = off // BT
    return pl.pallas_call(
        _block1_body,
        grid=(nrows // BT,),
        in_specs=[
            pl.BlockSpec((BT, D), lambda r: (r + off_t, 0)),
            pl.BlockSpec((D, N), lambda r: (0, 0)),
            pl.BlockSpec((N,), lambda r: (0,)),
            pl.BlockSpec((N, 1), lambda r: (0, 0)),
        ],
        out_specs=[
            pl.BlockSpec((BT, N), lambda r: (r, 0)),
            pl.BlockSpec((BT, 1), lambda r: (r, 0)),
        ],
        out_shape=[
            jax.ShapeDtypeStruct((nrows, N), jnp.float32),
            jax.ShapeDtypeStruct((nrows, 1), jnp.float32),
        ],
    )(x, w, b, g)


def _route_body(B, gate_ref, apos_ref, nr_ref):
    t = (gate_ref[...] > 0.0).astype(jnp.float32)
    C = t.shape[1]
    incl = (lax.broadcasted_iota(jnp.int32, (C, C), 0)
            <= lax.broadcasted_iota(jnp.int32, (C, C), 1)).astype(jnp.float32)
    cs = jnp.dot(t, incl, preferred_element_type=jnp.float32)
    rowtot = cs[:, C - 1:C]
    R = t.shape[0]
    tril = (lax.broadcasted_iota(jnp.int32, (R, R), 0)
            > lax.broadcasted_iota(jnp.int32, (R, R), 1)).astype(jnp.float32)
    rowoff = jnp.dot(tril, rowtot, preferred_element_type=jnp.float32)
    csg = cs + rowoff
    gidx = (lax.broadcasted_iota(jnp.int32, t.shape, 0) * 128
            + lax.broadcasted_iota(jnp.int32, t.shape, 1)).astype(jnp.float32)
    apos = jnp.where(t > 0.0, (B - 1) - (csg - 1.0), gidx - csg)
    apos_ref[...] = apos.astype(jnp.int32)
    nr_ref[...] = (B - (rowoff[R - 1:R, 0:1]
                        + rowtot[R - 1:R, 0:1])).astype(jnp.int32)


def _route(gate2d):
    R, C = gate2d.shape
    B = R * C
    return pl.pallas_call(
        functools.partial(_route_body, B),
        grid=(1,),
        in_specs=[pl.BlockSpec((R, C), lambda i: (0, 0))],
        out_specs=[
            pl.BlockSpec((R, C), lambda i: (0, 0)),
            pl.BlockSpec((1, 1), lambda i: (0, 0)),
        ],
        out_shape=[
            jax.ShapeDtypeStruct((R, C), jnp.int32),
            jax.ShapeDtypeStruct((1, 1), jnp.int32),
        ],
    )(gate2d)


def _scatter_body(per_w, rpc, src_hbm, idx_hbm, out_hbm,
                  idx_v, buf0, buf1, sem0, sem1):
    wid = lax.axis_index("s") * 2 + lax.axis_index("c")
    nchunks = per_w // rpc
    crow0 = wid * nchunks
    pltpu.sync_copy(idx_hbm.at[pl.ds(crow0, nchunks)], idx_v)
    bufs = (buf0, buf1)
    sems = (sem0, sem1)
    base = wid * per_w

    pltpu.sync_copy(src_hbm.at[pl.ds(base, rpc)], buf0)
    handles = [None, None]
    for c in range(nchunks):
        b = c & 1
        handles[b] = pltpu.async_copy(bufs[b], out_hbm.at[idx_v.at[c]],
                                      sems[b])
        if c + 1 < nchunks:
            nb = (c + 1) & 1
            if handles[nb] is not None:
                handles[nb].wait()
            pltpu.sync_copy(src_hbm.at[pl.ds(base + (c + 1) * rpc, rpc)],
                            bufs[nb])
    handles[(nchunks - 1) & 1].wait()
    if nchunks > 1:
        handles[nchunks & 1].wait()


def _scatter_rows(src, idx2d, rpc=16):
    B, D = src.shape
    n_w = 32
    per_w = B // n_w
    mesh = plsc.VectorSubcoreMesh(core_axis_name="c", subcore_axis_name="s")
    return pl.kernel(
        functools.partial(_scatter_body, per_w, rpc),
        out_type=jax.ShapeDtypeStruct((B, D), src.dtype),
        mesh=mesh,
        scratch_types=[
            pltpu.VMEM((per_w // rpc, rpc), jnp.int32),
            pltpu.VMEM((rpc, D), src.dtype),
            pltpu.VMEM((rpc, D), src.dtype),
            pltpu.SemaphoreType.DMA,
            pltpu.SemaphoreType.DMA,
        ],
    )(src, idx2d)


def _gather_body(per_w, rpc, src_hbm, idx_hbm, out_hbm,
                 idx_v, buf0, buf1, sem0, sem1):
    wid = lax.axis_index("s") * 2 + lax.axis_index("c")
    base = wid * per_w
    pltpu.sync_copy(idx_hbm.at[pl.ds(base, per_w)], idx_v)
    bufs = (buf0, buf1)
    sems = (sem0, sem1)
    nchunks = per_w // rpc

    handles = [None, None]
    handles[0] = pltpu.async_copy(src_hbm.at[idx_v.at[pl.ds(0, rpc)]],
                                  buf0, sem0)
    for c in range(nchunks):
        b = c & 1
        if c + 1 < nchunks:
            nb = (c + 1) & 1
            handles[nb] = pltpu.async_copy(
                src_hbm.at[idx_v.at[pl.ds((c + 1) * rpc, rpc)]],
                bufs[nb], sems[nb])
        handles[b].wait()
        pltpu.sync_copy(bufs[b], out_hbm.at[pl.ds(base + c * rpc, rpc)])


def _gather_rows(src, idx, rpc=32):
    B, D = src.shape
    n_w = 32
    per_w = B // n_w
    mesh = plsc.VectorSubcoreMesh(core_axis_name="c", subcore_axis_name="s")
    return pl.kernel(
        functools.partial(_gather_body, per_w, rpc),
        out_type=jax.ShapeDtypeStruct((B, D), src.dtype),
        mesh=mesh,
        scratch_types=[
            pltpu.VMEM((per_w,), jnp.int32),
            pltpu.VMEM((rpc, D), src.dtype),
            pltpu.VMEM((rpc, D), src.dtype),
            pltpu.SemaphoreType.DMA,
            pltpu.SemaphoreType.DMA,
        ],
    )(src, idx)


def _k23_body(nr_ref, h1c_ref, w2_ref, b2_ref, g2_ref,
              we1_ref, be1_ref, we2_ref, be2_ref, w3_ref, b3_ref, y_ref):
    r = pl.program_id(0)
    nr = nr_ref[0]
    row0 = r * BT
    rows = row0 + lax.broadcasted_iota(jnp.int32, (BT, 1), 0)
    x = h1c_ref[...]

    @pl.when(row0 + BT > nr)
    def _():
        y_ref[...] = (
            jnp.dot(x.astype(jnp.bfloat16), we1_ref[...],
                    preferred_element_type=jnp.float32) + be1_ref[...])

    @pl.when(row0 < nr)
    def _():
        h2 = jnp.dot(x, w2_ref[...], preferred_element_type=jnp.float32)
        h2 = jnp.maximum(h2 + b2_ref[...], 0.0)
        gate2 = jnp.dot(h2, g2_ref[...], preferred_element_type=jnp.float32)
        h2b = h2.astype(jnp.bfloat16)
        y2 = (jnp.dot(h2b, we2_ref[...], preferred_element_type=jnp.float32)
              + be2_ref[...])
        y3 = (jnp.dot(h2b, w3_ref[...], preferred_element_type=jnp.float32)
              + b3_ref[...])
        y23 = jnp.where(gate2 > 0.0, y2, y3)
        y_ref[...] = jnp.where(rows < nr, y23, y_ref[...])


def _k23(nr, h1c, w2, b2, g2, we1, be1, we2, be2, w3, b3):
    B, D = h1c.shape
    N = we1.shape[1]
    Npad = (N + 127) // 128 * 128
    wspec = pl.BlockSpec((D, N), lambda r, s: (0, 0))
    bspec = pl.BlockSpec((1, N), lambda r, s: (0, 0))
    return pl.pallas_call(
        _k23_body,
        grid_spec=pltpu.PrefetchScalarGridSpec(
            num_scalar_prefetch=1,
            grid=(B // BT,),
            in_specs=[
                pl.BlockSpec((BT, D), lambda r, s: (r, 0)),
                pl.BlockSpec((D, D), lambda r, s: (0, 0)),
                pl.BlockSpec((D,), lambda r, s: (0,)),
                pl.BlockSpec((D, 1), lambda r, s: (0, 0)),
                wspec, bspec, wspec, bspec, wspec, bspec,
            ],
            out_specs=pl.BlockSpec((BT, Npad), lambda r, s: (r, 0)),
        ),
        out_shape=jax.ShapeDtypeStruct((B, Npad), jnp.float32),
    )(nr, h1c, w2, b2, g2, we1, be1, we2, be2, w3, b3)


def kernel(X, W1, b1, g1, We1, be1, W2, b2, g2, We2, be2, W3, b3):
    B = X.shape[0]
    O = be1.shape[0]

    Opad = (O + 127) // 128 * 128
    pc = ((0, 0), (0, Opad - O))
    we1 = jnp.pad(We1, pc).astype(jnp.bfloat16)
    we2 = jnp.pad(We2, pc).astype(jnp.bfloat16)
    w3 = jnp.pad(W3, pc).astype(jnp.bfloat16)
    bp = (0, Opad - O)

    h1, gate1 = _block1(X, W1, b1, g1, 0, B)
    apos2d, nr11 = _route(gate1.reshape(B // 128, 128))
    apos = apos2d.reshape(B)
    h1c = _scatter_rows(h1, apos.reshape(B // 16, 16), rpc=16)
    yc = _k23(nr11.reshape(1), h1c, W2, b2, g2,
              we1, jnp.pad(be1, bp).reshape(1, Opad),
              we2, jnp.pad(be2, bp).reshape(1, Opad),
              w3, jnp.pad(b3, bp).reshape(1, Opad))
    y = _gather_rows(yc, apos, rpc=32)
    return y[:, :O]


# speedup vs baseline: 1.1043x; 1.1043x over previous
import functools

import jax
import jax.numpy as jnp
from jax import lax
from jax.experimental import pallas as pl
from jax.experimental.pallas import tpu as pltpu
from jax.experimental.pallas import tpu_sc as plsc

BT = 256
_L = 16


def _block1_body(x_ref, w_ref, b_ref, g_ref, h_ref, gate_ref):
    h = jnp.dot(x_ref[...], w_ref[...], preferred_element_type=jnp.float32)
    h = jnp.maximum(h + b_ref[...][None, :], 0.0)
    h_ref[...] = h
    gate_ref[...] = jnp.dot(h, g_ref[...], preferred_element_type=jnp.float32)


def _block1(x, w, b, g, off, nrows):
    B, D = x.shape
    N = w.shape[1]
    off_t = off // BT
    return pl.pallas_call(
        _block1_body,
        grid=(nrows // BT,),
        in_specs=[
            pl.BlockSpec((BT, D), lambda r: (r + off_t, 0)),
            pl.BlockSpec((D, N), lambda r: (0, 0)),
            pl.BlockSpec((N,), lambda r: (0,)),
            pl.BlockSpec((N, 1), lambda r: (0, 0)),
        ],
        out_specs=[
            pl.BlockSpec((BT, N), lambda r: (r, 0)),
            pl.BlockSpec((BT, 1), lambda r: (r, 0)),
        ],
        out_shape=[
            jax.ShapeDtypeStruct((nrows, N), jnp.float32),
            jax.ShapeDtypeStruct((nrows, 1), jnp.float32),
        ],
    )(x, w, b, g)


def _route_body(B, gate_ref, apos_ref, nr_ref):
    t = (gate_ref[...] > 0.0).astype(jnp.float32)
    C = t.shape[1]
    incl = (lax.broadcasted_iota(jnp.int32, (C, C), 0)
            <= lax.broadcasted_iota(jnp.int32, (C, C), 1)).astype(jnp.float32)
    cs = jnp.dot(t, incl, preferred_element_type=jnp.float32)
    rowtot = cs[:, C - 1:C]
    R = t.shape[0]
    tril = (lax.broadcasted_iota(jnp.int32, (R, R), 0)
            > lax.broadcasted_iota(jnp.int32, (R, R), 1)).astype(jnp.float32)
    rowoff = jnp.dot(tril, rowtot, preferred_element_type=jnp.float32)
    csg = cs + rowoff
    gidx = (lax.broadcasted_iota(jnp.int32, t.shape, 0) * 128
            + lax.broadcasted_iota(jnp.int32, t.shape, 1)).astype(jnp.float32)
    apos = jnp.where(t > 0.0, (B - 1) - (csg - 1.0), gidx - csg)
    apos_ref[...] = apos.astype(jnp.int32)
    nr_ref[...] = (B - (rowoff[R - 1:R, 0:1]
                        + rowtot[R - 1:R, 0:1])).astype(jnp.int32)


def _route(gate2d):
    R, C = gate2d.shape
    B = R * C
    return pl.pallas_call(
        functools.partial(_route_body, B),
        grid=(1,),
        in_specs=[pl.BlockSpec((R, C), lambda i: (0, 0))],
        out_specs=[
            pl.BlockSpec((R, C), lambda i: (0, 0)),
            pl.BlockSpec((1, 1), lambda i: (0, 0)),
        ],
        out_shape=[
            jax.ShapeDtypeStruct((R, C), jnp.int32),
            jax.ShapeDtypeStruct((1, 1), jnp.int32),
        ],
    )(gate2d)


def _scatter_body(per_w, rpc, src_hbm, idx_hbm, out_hbm,
                  idx_v, buf0, buf1, sem0, sem1):
    wid = lax.axis_index("s") * 2 + lax.axis_index("c")
    nchunks = per_w // rpc
    crow0 = wid * nchunks
    pltpu.sync_copy(idx_hbm.at[pl.ds(crow0, nchunks)], idx_v)
    bufs = (buf0, buf1)
    sems = (sem0, sem1)
    base = wid * per_w

    pltpu.sync_copy(src_hbm.at[pl.ds(base, rpc)], buf0)
    handles = [None, None]
    for c in range(nchunks):
        b = c & 1
        handles[b] = pltpu.async_copy(bufs[b], out_hbm.at[idx_v.at[c]],
                                      sems[b])
        if c + 1 < nchunks:
            nb = (c + 1) & 1
            if handles[nb] is not None:
                handles[nb].wait()
            pltpu.sync_copy(src_hbm.at[pl.ds(base + (c + 1) * rpc, rpc)],
                            bufs[nb])
    handles[(nchunks - 1) & 1].wait()
    if nchunks > 1:
        handles[nchunks & 1].wait()


def _scatter_rows(src, idx2d, rpc=16):
    B, D = src.shape
    n_w = 32
    per_w = B // n_w
    mesh = plsc.VectorSubcoreMesh(core_axis_name="c", subcore_axis_name="s")
    return pl.kernel(
        functools.partial(_scatter_body, per_w, rpc),
        out_type=jax.ShapeDtypeStruct((B, D), src.dtype),
        mesh=mesh,
        scratch_types=[
            pltpu.VMEM((per_w // rpc, rpc), jnp.int32),
            pltpu.VMEM((rpc, D), src.dtype),
            pltpu.VMEM((rpc, D), src.dtype),
            pltpu.SemaphoreType.DMA,
            pltpu.SemaphoreType.DMA,
        ],
    )(src, idx2d)


def _gather_body(per_w, rpc, src_hbm, idx_hbm, out_hbm,
                 idx_v, buf0, buf1, sem0, sem1):
    wid = lax.axis_index("s") * 2 + lax.axis_index("c")
    base = wid * per_w
    pltpu.sync_copy(idx_hbm.at[pl.ds(base, per_w)], idx_v)
    bufs = (buf0, buf1)
    sems = (sem0, sem1)
    nchunks = per_w // rpc

    handles = [None, None]
    handles[0] = pltpu.async_copy(src_hbm.at[idx_v.at[pl.ds(0, rpc)]],
                                  buf0, sem0)
    for c in range(nchunks):
        b = c & 1
        if c + 1 < nchunks:
            nb = (c + 1) & 1
            handles[nb] = pltpu.async_copy(
                src_hbm.at[idx_v.at[pl.ds((c + 1) * rpc, rpc)]],
                bufs[nb], sems[nb])
        handles[b].wait()
        pltpu.sync_copy(bufs[b], out_hbm.at[pl.ds(base + c * rpc, rpc)])


def _gather_rows(src, idx, rpc=32):
    B = src.shape[0]
    D = src.shape[1:]
    n_w = 32
    per_w = B // n_w
    mesh = plsc.VectorSubcoreMesh(core_axis_name="c", subcore_axis_name="s")
    return pl.kernel(
        functools.partial(_gather_body, per_w, rpc),
        out_type=jax.ShapeDtypeStruct((B,) + D, src.dtype),
        mesh=mesh,
        scratch_types=[
            pltpu.VMEM((per_w,), jnp.int32),
            pltpu.VMEM((rpc,) + D, src.dtype),
            pltpu.VMEM((rpc,) + D, src.dtype),
            pltpu.SemaphoreType.DMA,
            pltpu.SemaphoreType.DMA,
        ],
    )(src, idx)


def _k23_body(nr_ref, h1c_ref, w2_ref, b2_ref, g2_ref,
              we1_ref, be1_ref, we2_ref, be2_ref, w3_ref, b3_ref, y_ref):
    r = pl.program_id(0)
    nr = nr_ref[0]
    row0 = r * BT
    rows = row0 + lax.broadcasted_iota(jnp.int32, (BT, 1), 0)
    x = h1c_ref[...]

    N = we1_ref.shape[1]

    @pl.when(row0 + BT > nr)
    def _():
        y_ref[:, pl.ds(0, N)] = (
            jnp.dot(x.astype(jnp.bfloat16), we1_ref[...],
                    preferred_element_type=jnp.float32) + be1_ref[...])

    @pl.when(row0 < nr)
    def _():
        h2 = jnp.dot(x, w2_ref[...], preferred_element_type=jnp.float32)
        h2 = jnp.maximum(h2 + b2_ref[...], 0.0)
        gate2 = jnp.dot(h2, g2_ref[...], preferred_element_type=jnp.float32)
        h2b = h2.astype(jnp.bfloat16)
        y2 = (jnp.dot(h2b, we2_ref[...], preferred_element_type=jnp.float32)
              + be2_ref[...])
        y3 = (jnp.dot(h2b, w3_ref[...], preferred_element_type=jnp.float32)
              + b3_ref[...])
        y23 = jnp.where(gate2 > 0.0, y2, y3)
        y_ref[:, pl.ds(0, N)] = jnp.where(rows < nr, y23,
                                          y_ref[:, pl.ds(0, N)])


def _k23(nr, h1c, w2, b2, g2, we1, be1, we2, be2, w3, b3):
    B, D = h1c.shape
    N = we1.shape[1]
    Npad = (N + 127) // 128 * 128
    wspec = pl.BlockSpec((D, N), lambda r, s: (0, 0))
    bspec = pl.BlockSpec((1, N), lambda r, s: (0, 0))
    return pl.pallas_call(
        _k23_body,
        grid_spec=pltpu.PrefetchScalarGridSpec(
            num_scalar_prefetch=1,
            grid=(B // BT,),
            in_specs=[
                pl.BlockSpec((BT, D), lambda r, s: (r, 0)),
                pl.BlockSpec((D, D), lambda r, s: (0, 0)),
                pl.BlockSpec((D,), lambda r, s: (0,)),
                pl.BlockSpec((D, 1), lambda r, s: (0, 0)),
                wspec, bspec, wspec, bspec, wspec, bspec,
            ],
            out_specs=pl.BlockSpec((BT, Npad), lambda r, s: (r, 0)),
        ),
        out_shape=jax.ShapeDtypeStruct((B, Npad), jnp.float32),
    )(nr, h1c, w2, b2, g2, we1, be1, we2, be2, w3, b3)


def kernel(X, W1, b1, g1, We1, be1, W2, b2, g2, We2, be2, W3, b3):
    B = X.shape[0]
    O = be1.shape[0]

    we1 = We1.astype(jnp.bfloat16)
    we2 = We2.astype(jnp.bfloat16)
    w3 = W3.astype(jnp.bfloat16)

    h1, gate1 = _block1(X, W1, b1, g1, 0, B)
    apos2d, nr11 = _route(gate1.reshape(B // 128, 128))
    apos = apos2d.reshape(B)
    h1c = _scatter_rows(h1, apos.reshape(B // 16, 16), rpc=16)
    yc = _k23(nr11.reshape(1), h1c, W2, b2, g2,
              we1, be1.reshape(1, O), we2, be2.reshape(1, O),
              w3, b3.reshape(1, O))
    y = _gather_rows(yc, apos, rpc=32)
    return y[:, :O]


# speedup vs baseline: 1.1080x; 1.0033x over previous
import functools

import jax
import jax.numpy as jnp
from jax import lax
from jax.experimental import pallas as pl
from jax.experimental.pallas import tpu as pltpu
from jax.experimental.pallas import tpu_sc as plsc

BT = 512
_L = 16


def _block1_body(x_ref, w_ref, b_ref, g_ref, h_ref, gate_ref):
    h = jnp.dot(x_ref[...], w_ref[...], preferred_element_type=jnp.float32)
    h = jnp.maximum(h + b_ref[...][None, :], 0.0)
    h_ref[...] = h
    gate_ref[...] = jnp.dot(h, g_ref[...], preferred_element_type=jnp.float32)


def _block1(x, w, b, g, off, nrows):
    B, D = x.shape
    N = w.shape[1]
    off_t = off // BT
    return pl.pallas_call(
        _block1_body,
        grid=(nrows // BT,),
        in_specs=[
            pl.BlockSpec((BT, D), lambda r: (r + off_t, 0)),
            pl.BlockSpec((D, N), lambda r: (0, 0)),
            pl.BlockSpec((N,), lambda r: (0,)),
            pl.BlockSpec((N, 1), lambda r: (0, 0)),
        ],
        out_specs=[
            pl.BlockSpec((BT, N), lambda r: (r, 0)),
            pl.BlockSpec((BT, 1), lambda r: (r, 0)),
        ],
        out_shape=[
            jax.ShapeDtypeStruct((nrows, N), jnp.float32),
            jax.ShapeDtypeStruct((nrows, 1), jnp.float32),
        ],
    )(x, w, b, g)


def _route_body(B, gate_ref, apos_ref, nr_ref):
    t = (gate_ref[...] > 0.0).astype(jnp.float32)
    C = t.shape[1]
    incl = (lax.broadcasted_iota(jnp.int32, (C, C), 0)
            <= lax.broadcasted_iota(jnp.int32, (C, C), 1)).astype(jnp.float32)
    cs = jnp.dot(t, incl, preferred_element_type=jnp.float32)
    rowtot = cs[:, C - 1:C]
    R = t.shape[0]
    tril = (lax.broadcasted_iota(jnp.int32, (R, R), 0)
            > lax.broadcasted_iota(jnp.int32, (R, R), 1)).astype(jnp.float32)
    rowoff = jnp.dot(tril, rowtot, preferred_element_type=jnp.float32)
    csg = cs + rowoff
    gidx = (lax.broadcasted_iota(jnp.int32, t.shape, 0) * 128
            + lax.broadcasted_iota(jnp.int32, t.shape, 1)).astype(jnp.float32)
    apos = jnp.where(t > 0.0, (B - 1) - (csg - 1.0), gidx - csg)
    apos_ref[...] = apos.astype(jnp.int32)
    nr_ref[...] = (B - (rowoff[R - 1:R, 0:1]
                        + rowtot[R - 1:R, 0:1])).astype(jnp.int32)


def _route(gate2d):
    R, C = gate2d.shape
    B = R * C
    return pl.pallas_call(
        functools.partial(_route_body, B),
        grid=(1,),
        in_specs=[pl.BlockSpec((R, C), lambda i: (0, 0))],
        out_specs=[
            pl.BlockSpec((R, C), lambda i: (0, 0)),
            pl.BlockSpec((1, 1), lambda i: (0, 0)),
        ],
        out_shape=[
            jax.ShapeDtypeStruct((R, C), jnp.int32),
            jax.ShapeDtypeStruct((1, 1), jnp.int32),
        ],
    )(gate2d)


def _scatter_body(per_w, rpc, src_hbm, idx_hbm, out_hbm,
                  idx_v, buf0, buf1, sem0, sem1):
    wid = lax.axis_index("s") * 2 + lax.axis_index("c")
    nchunks = per_w // rpc
    crow0 = wid * nchunks
    pltpu.sync_copy(idx_hbm.at[pl.ds(crow0, nchunks)], idx_v)
    bufs = (buf0, buf1)
    sems = (sem0, sem1)
    base = wid * per_w

    pltpu.sync_copy(src_hbm.at[pl.ds(base, rpc)], buf0)
    handles = [None, None]
    for c in range(nchunks):
        b = c & 1
        handles[b] = pltpu.async_copy(bufs[b], out_hbm.at[idx_v.at[c]],
                                      sems[b])
        if c + 1 < nchunks:
            nb = (c + 1) & 1
            if handles[nb] is not None:
                handles[nb].wait()
            pltpu.sync_copy(src_hbm.at[pl.ds(base + (c + 1) * rpc, rpc)],
                            bufs[nb])
    handles[(nchunks - 1) & 1].wait()
    if nchunks > 1:
        handles[nchunks & 1].wait()


def _scatter_rows(src, idx2d, rpc=16):
    B, D = src.shape
    n_w = 32
    per_w = B // n_w
    mesh = plsc.VectorSubcoreMesh(core_axis_name="c", subcore_axis_name="s")
    return pl.kernel(
        functools.partial(_scatter_body, per_w, rpc),
        out_type=jax.ShapeDtypeStruct((B, D), src.dtype),
        mesh=mesh,
        scratch_types=[
            pltpu.VMEM((per_w // rpc, rpc), jnp.int32),
            pltpu.VMEM((rpc, D), src.dtype),
            pltpu.VMEM((rpc, D), src.dtype),
            pltpu.SemaphoreType.DMA,
            pltpu.SemaphoreType.DMA,
        ],
    )(src, idx2d)


def _gather_body(per_w, rpc, src_hbm, idx_hbm, out_hbm,
                 idx_v, buf0, buf1, sem0, sem1):
    wid = lax.axis_index("s") * 2 + lax.axis_index("c")
    base = wid * per_w
    pltpu.sync_copy(idx_hbm.at[pl.ds(base, per_w)], idx_v)
    bufs = (buf0, buf1)
    sems = (sem0, sem1)
    nchunks = per_w // rpc

    handles = [None, None]
    handles[0] = pltpu.async_copy(src_hbm.at[idx_v.at[pl.ds(0, rpc)]],
                                  buf0, sem0)
    for c in range(nchunks):
        b = c & 1
        if c + 1 < nchunks:
            nb = (c + 1) & 1
            handles[nb] = pltpu.async_copy(
                src_hbm.at[idx_v.at[pl.ds((c + 1) * rpc, rpc)]],
                bufs[nb], sems[nb])
        handles[b].wait()
        pltpu.sync_copy(bufs[b], out_hbm.at[pl.ds(base + c * rpc, rpc)])


def _gather_rows(src, idx, rpc=32):
    B = src.shape[0]
    D = src.shape[1:]
    n_w = 32
    per_w = B // n_w
    mesh = plsc.VectorSubcoreMesh(core_axis_name="c", subcore_axis_name="s")
    return pl.kernel(
        functools.partial(_gather_body, per_w, rpc),
        out_type=jax.ShapeDtypeStruct((B,) + D, src.dtype),
        mesh=mesh,
        scratch_types=[
            pltpu.VMEM((per_w,), jnp.int32),
            pltpu.VMEM((rpc,) + D, src.dtype),
            pltpu.VMEM((rpc,) + D, src.dtype),
            pltpu.SemaphoreType.DMA,
            pltpu.SemaphoreType.DMA,
        ],
    )(src, idx)


def _k23_body(nr_ref, h1c_ref, w2_ref, b2_ref, g2_ref,
              we1_ref, be1_ref, we2_ref, be2_ref, w3_ref, b3_ref, y_ref):
    r = pl.program_id(0)
    nr = nr_ref[0]
    row0 = r * BT
    rows = row0 + lax.broadcasted_iota(jnp.int32, (BT, 1), 0)
    x = h1c_ref[...]

    N = we1_ref.shape[1]

    @pl.when(row0 + BT > nr)
    def _():
        y_ref[:, pl.ds(0, N)] = (
            jnp.dot(x.astype(jnp.bfloat16), we1_ref[...],
                    preferred_element_type=jnp.float32) + be1_ref[...])

    @pl.when(row0 < nr)
    def _():
        h2 = jnp.dot(x, w2_ref[...], preferred_element_type=jnp.float32)
        h2 = jnp.maximum(h2 + b2_ref[...], 0.0)
        gate2 = jnp.dot(h2, g2_ref[...], preferred_element_type=jnp.float32)
        h2b = h2.astype(jnp.bfloat16)
        y2 = (jnp.dot(h2b, we2_ref[...], preferred_element_type=jnp.float32)
              + be2_ref[...])
        y3 = (jnp.dot(h2b, w3_ref[...], preferred_element_type=jnp.float32)
              + b3_ref[...])
        y23 = jnp.where(gate2 > 0.0, y2, y3)
        y_ref[:, pl.ds(0, N)] = jnp.where(rows < nr, y23,
                                          y_ref[:, pl.ds(0, N)])


def _k23(nr, h1c, w2, b2, g2, we1, be1, we2, be2, w3, b3):
    B, D = h1c.shape
    N = we1.shape[1]
    Npad = (N + 127) // 128 * 128
    wspec = pl.BlockSpec((D, N), lambda r, s: (0, 0))
    bspec = pl.BlockSpec((1, N), lambda r, s: (0, 0))
    return pl.pallas_call(
        _k23_body,
        grid_spec=pltpu.PrefetchScalarGridSpec(
            num_scalar_prefetch=1,
            grid=(B // BT,),
            in_specs=[
                pl.BlockSpec((BT, D), lambda r, s: (r, 0)),
                pl.BlockSpec((D, D), lambda r, s: (0, 0)),
                pl.BlockSpec((D,), lambda r, s: (0,)),
                pl.BlockSpec((D, 1), lambda r, s: (0, 0)),
                wspec, bspec, wspec, bspec, wspec, bspec,
            ],
            out_specs=pl.BlockSpec((BT, Npad), lambda r, s: (r, 0)),
        ),
        out_shape=jax.ShapeDtypeStruct((B, Npad), jnp.float32),
    )(nr, h1c, w2, b2, g2, we1, be1, we2, be2, w3, b3)


def kernel(X, W1, b1, g1, We1, be1, W2, b2, g2, We2, be2, W3, b3):
    B = X.shape[0]
    O = be1.shape[0]

    we1 = We1.astype(jnp.bfloat16)
    we2 = We2.astype(jnp.bfloat16)
    w3 = W3.astype(jnp.bfloat16)

    h1, gate1 = _block1(X, W1, b1, g1, 0, B)
    apos2d, nr11 = _route(gate1.reshape(B // 128, 128))
    apos = apos2d.reshape(B)
    h1c = _scatter_rows(h1, apos.reshape(B // 16, 16), rpc=16)
    yc = _k23(nr11.reshape(1), h1c, W2, b2, g2,
              we1, be1.reshape(1, O), we2, be2.reshape(1, O),
              w3, b3.reshape(1, O))
    y = _gather_rows(yc, apos, rpc=32)
    return y[:, :O]
